# Initial kernel scaffold; baseline (speedup 1.0000x reference)
import jax
import jax.numpy as jnp
from jax.experimental import pallas as pl


def kernel(x, pos, batch, weight, bias):
    raise NotImplementedError("write your pallas kernel here")


# speedup vs baseline: 11.1844x; 11.1844x over previous
import functools

import jax
import jax.numpy as jnp
from jax import lax
from jax.experimental import pallas as pl
from jax.experimental.pallas import tpu as pltpu


NUM_GRAPHS_STATIC = 64
_NEG_INF = float("-inf")


def _round_up(v, m):
    return ((v + m - 1) // m) * m


def _fused_kernel(offs_ref, x_ref, w_ref, b_ref, o_ref,
                  y_ref, bm_ref, acc_ref,
                  *, tm, rb, n_tiles, tiles_per_core, num_graphs):
    c = pl.program_id(0)
    i = pl.program_id(1)
    nb = tm // rb

    @pl.when(i == 0)
    def _init():
        acc_ref[...] = jnp.full(acc_ref.shape, _NEG_INF, dtype=acc_ref.dtype)

    y_ref[...] = lax.dot_general(
        x_ref[...].astype(jnp.bfloat16), w_ref[...].astype(jnp.bfloat16),
        dimension_numbers=(((1,), (1,)), ((), ())),
        preferred_element_type=jnp.float32,
    )

    for j in range(nb):
        bm_ref[j:j + 1, :] = y_ref[j * rb:(j + 1) * rb, :].max(
            axis=0, keepdims=True)

    tile_idx = jnp.minimum(c * tiles_per_core + i, n_tiles - 1)
    ts = tile_idx * tm
    te = ts + tm

    block_rows = lax.broadcasted_iota(jnp.int32, (nb, 1), 0)
    part_rows = lax.broadcasted_iota(jnp.int32, (rb, 1), 0)

    for g in range(num_graphs):
        lo = offs_ref[g]
        hi = offs_ref[g + 1]

        @pl.when(jnp.logical_and(hi > ts, lo < te))
        def _graph(g=g, lo=lo, hi=hi):
            a = jnp.maximum(lo, ts) - ts
            b = jnp.minimum(hi, te) - ts
            j_first = a // rb
            j_last = (b - 1) // rb
            full_lo = (a + rb - 1) // rb
            full_hi = b // rb

            bm_mask = jnp.logical_and(block_rows >= full_lo,
                                      block_rows < full_hi)
            cand = jnp.where(bm_mask, bm_ref[...], _NEG_INF).max(
                axis=0, keepdims=True)
            acc_ref[g:g + 1, :] = jnp.maximum(acc_ref[g:g + 1, :], cand)

            need_a = jnp.logical_or(
                a % rb != 0,
                jnp.logical_and(j_first == j_last, b % rb != 0))

            @pl.when(need_a)
            def _partial_a():
                blk = y_ref[pl.ds(j_first * rb, rb), :]
                rows = j_first * rb + part_rows
                m = jnp.logical_and(rows >= a, rows < b)
                pa = jnp.where(m, blk, _NEG_INF).max(axis=0, keepdims=True)
                acc_ref[g:g + 1, :] = jnp.maximum(acc_ref[g:g + 1, :], pa)

            need_b = jnp.logical_and(b % rb != 0, j_last > j_first)

            @pl.when(need_b)
            def _partial_b():
                blk = y_ref[pl.ds(j_last * rb, rb), :]
                rows = j_last * rb + part_rows
                m = jnp.logical_and(rows >= a, rows < b)
                pb = jnp.where(m, blk, _NEG_INF).max(axis=0, keepdims=True)
                acc_ref[g:g + 1, :] = jnp.maximum(acc_ref[g:g + 1, :], pb)

    @pl.when(i == pl.num_programs(1) - 1)
    def _finalize():
        o_ref[...] = (acc_ref[...] + b_ref[...]).astype(o_ref.dtype)


def _forward(x, pos, batch, weight, bias, num_graphs, *, tm=4096, rb=64):
    n, in_c = x.shape
    out_c = weight.shape[0]

    out_c_pad = _round_up(out_c, 128)
    g_pad = _round_up(num_graphs, 8)
    if out_c_pad != out_c:
        weight = jnp.pad(weight, ((0, out_c_pad - out_c), (0, 0)))
        bias = jnp.pad(bias, (0, out_c_pad - out_c))
    b2d = bias.reshape(1, out_c_pad).astype(jnp.float32)

    offsets = jnp.searchsorted(
        batch.astype(jnp.int32),
        jnp.arange(num_graphs + 1, dtype=jnp.int32)).astype(jnp.int32)

    tm_eff = min(max(rb, (tm // rb) * rb), _round_up(n, rb))
    n_tiles = pl.cdiv(n, tm_eff)
    num_cores = 2 if n_tiles >= 2 else 1
    tiles_per_core = pl.cdiv(n_tiles, num_cores)

    def x_map(c, i, offs):
        return (jnp.minimum(c * tiles_per_core + i, n_tiles - 1), 0)

    kernel_fn = functools.partial(
        _fused_kernel,
        tm=tm_eff, rb=rb, n_tiles=n_tiles, tiles_per_core=tiles_per_core,
        num_graphs=num_graphs)

    bytes_accessed = (x.size * x.dtype.itemsize
                      + weight.size * weight.dtype.itemsize
                      + (num_graphs + 1) * 4
                      + num_cores * g_pad * out_c_pad * 4)

    out = pl.pallas_call(
        kernel_fn,
        out_shape=jax.ShapeDtypeStruct((num_cores, g_pad, out_c_pad),
                                       jnp.float32),
        grid_spec=pltpu.PrefetchScalarGridSpec(
            num_scalar_prefetch=1,
            grid=(num_cores, tiles_per_core),
            in_specs=[
                pl.BlockSpec((tm_eff, in_c), x_map),
                pl.BlockSpec((out_c_pad, in_c), lambda c, i, offs: (0, 0)),
                pl.BlockSpec((1, out_c_pad), lambda c, i, offs: (0, 0)),
            ],
            out_specs=pl.BlockSpec((None, g_pad, out_c_pad),
                                   lambda c, i, offs: (c, 0, 0)),
            scratch_shapes=[
                pltpu.VMEM((tm_eff, out_c_pad), jnp.float32),
                pltpu.VMEM((tm_eff // rb, out_c_pad), jnp.float32),
                pltpu.VMEM((g_pad, out_c_pad), jnp.float32),
            ],
        ),
        compiler_params=pltpu.CompilerParams(
            dimension_semantics=("parallel", "arbitrary"),
            vmem_limit_bytes=48 * 1024 * 1024,
        ),
        cost_estimate=pl.CostEstimate(
            flops=2 * n * in_c * out_c_pad,
            transcendentals=0,
            bytes_accessed=bytes_accessed,
        ),
    )(offsets, x, weight, b2d)

    pooled = jnp.max(out, axis=0)[:num_graphs, :out_c]
    pos_out = jnp.zeros((num_graphs, 3), dtype=pos.dtype)
    batch_out = jnp.arange(num_graphs, dtype=jnp.int32)
    return pooled, pos_out, batch_out


def kernel(x, pos, batch, weight, bias):
    return _forward(x, pos, batch, weight, bias, NUM_GRAPHS_STATIC)


# speedup vs baseline: 11.2035x; 1.0017x over previous
import functools

import jax
import jax.numpy as jnp
from jax import lax
from jax.experimental import pallas as pl
from jax.experimental.pallas import tpu as pltpu


NUM_GRAPHS_STATIC = 64
_NEG_INF = float("-inf")


def _round_up(v, m):
    return ((v + m - 1) // m) * m


def _fused_kernel(offs_ref, x_ref, w_ref, b_ref, o_ref,
                  y_ref, bm_ref, acc_ref,
                  *, tm, rb, n_tiles, tiles_per_core, num_graphs):
    c = pl.program_id(0)
    i = pl.program_id(1)
    nb = tm // rb

    @pl.when(i == 0)
    def _init():
        acc_ref[...] = jnp.full(acc_ref.shape, _NEG_INF, dtype=acc_ref.dtype)

    y_ref[...] = lax.dot_general(
        x_ref[...].astype(jnp.bfloat16), w_ref[...].astype(jnp.bfloat16),
        dimension_numbers=(((1,), (1,)), ((), ())),
        preferred_element_type=jnp.float32,
    )

    for j in range(nb):
        bm_ref[j:j + 1, :] = y_ref[j * rb:(j + 1) * rb, :].max(
            axis=0, keepdims=True)

    tile_idx = jnp.minimum(c * tiles_per_core + i, n_tiles - 1)
    ts = tile_idx * tm
    te = ts + tm

    block_rows = lax.broadcasted_iota(jnp.int32, (nb, 1), 0)
    part_rows = lax.broadcasted_iota(jnp.int32, (rb, 1), 0)

    for g in range(num_graphs):
        lo = offs_ref[g]
        hi = offs_ref[g + 1]

        @pl.when(jnp.logical_and(hi > ts, lo < te))
        def _graph(g=g, lo=lo, hi=hi):
            a = jnp.maximum(lo, ts) - ts
            b = jnp.minimum(hi, te) - ts
            j_first = a // rb
            j_last = (b - 1) // rb
            full_lo = (a + rb - 1) // rb
            full_hi = b // rb

            bm_mask = jnp.logical_and(block_rows >= full_lo,
                                      block_rows < full_hi)
            cand = jnp.where(bm_mask, bm_ref[...], _NEG_INF).max(
                axis=0, keepdims=True)
            acc_ref[g:g + 1, :] = jnp.maximum(acc_ref[g:g + 1, :], cand)

            need_a = jnp.logical_or(
                a % rb != 0,
                jnp.logical_and(j_first == j_last, b % rb != 0))

            @pl.when(need_a)
            def _partial_a():
                blk = y_ref[pl.ds(j_first * rb, rb), :]
                rows = j_first * rb + part_rows
                m = jnp.logical_and(rows >= a, rows < b)
                pa = jnp.where(m, blk, _NEG_INF).max(axis=0, keepdims=True)
                acc_ref[g:g + 1, :] = jnp.maximum(acc_ref[g:g + 1, :], pa)

            need_b = jnp.logical_and(b % rb != 0, j_last > j_first)

            @pl.when(need_b)
            def _partial_b():
                blk = y_ref[pl.ds(j_last * rb, rb), :]
                rows = j_last * rb + part_rows
                m = jnp.logical_and(rows >= a, rows < b)
                pb = jnp.where(m, blk, _NEG_INF).max(axis=0, keepdims=True)
                acc_ref[g:g + 1, :] = jnp.maximum(acc_ref[g:g + 1, :], pb)

    @pl.when(i == pl.num_programs(1) - 1)
    def _finalize():
        o_ref[...] = (acc_ref[...] + b_ref[...]).astype(o_ref.dtype)


def _forward(x, pos, batch, weight, bias, num_graphs, *, tm=4096, rb=64):
    n, in_c = x.shape
    out_c = weight.shape[0]

    out_c_pad = _round_up(out_c, 128)
    g_pad = _round_up(num_graphs, 8)
    if out_c_pad != out_c:
        weight = jnp.pad(weight, ((0, out_c_pad - out_c), (0, 0)))
        bias = jnp.pad(bias, (0, out_c_pad - out_c))
    b2d = bias.reshape(1, out_c_pad).astype(jnp.float32)

    offsets = jnp.searchsorted(
        batch.astype(jnp.int32),
        jnp.arange(num_graphs + 1, dtype=jnp.int32)).astype(jnp.int32)

    tm_eff = min(max(rb, (tm // rb) * rb), _round_up(n, rb))
    n_tiles = pl.cdiv(n, tm_eff)
    num_cores = 1
    tiles_per_core = pl.cdiv(n_tiles, num_cores)

    def x_map(c, i, offs):
        return (jnp.minimum(c * tiles_per_core + i, n_tiles - 1), 0)

    kernel_fn = functools.partial(
        _fused_kernel,
        tm=tm_eff, rb=rb, n_tiles=n_tiles, tiles_per_core=tiles_per_core,
        num_graphs=num_graphs)

    bytes_accessed = (x.size * x.dtype.itemsize
                      + weight.size * weight.dtype.itemsize
                      + (num_graphs + 1) * 4
                      + num_cores * g_pad * out_c_pad * 4)

    out = pl.pallas_call(
        kernel_fn,
        out_shape=jax.ShapeDtypeStruct((num_cores, g_pad, out_c_pad),
                                       jnp.float32),
        grid_spec=pltpu.PrefetchScalarGridSpec(
            num_scalar_prefetch=1,
            grid=(num_cores, tiles_per_core),
            in_specs=[
                pl.BlockSpec((tm_eff, in_c), x_map),
                pl.BlockSpec((out_c_pad, in_c), lambda c, i, offs: (0, 0)),
                pl.BlockSpec((1, out_c_pad), lambda c, i, offs: (0, 0)),
            ],
            out_specs=pl.BlockSpec((None, g_pad, out_c_pad),
                                   lambda c, i, offs: (c, 0, 0)),
            scratch_shapes=[
                pltpu.VMEM((tm_eff, out_c_pad), jnp.float32),
                pltpu.VMEM((tm_eff // rb, out_c_pad), jnp.float32),
                pltpu.VMEM((g_pad, out_c_pad), jnp.float32),
            ],
        ),
        compiler_params=pltpu.CompilerParams(
            dimension_semantics=("parallel", "arbitrary"),
            vmem_limit_bytes=48 * 1024 * 1024,
        ),
        cost_estimate=pl.CostEstimate(
            flops=2 * n * in_c * out_c_pad,
            transcendentals=0,
            bytes_accessed=bytes_accessed,
        ),
    )(offsets, x, weight, b2d)

    pooled = jnp.max(out, axis=0)[:num_graphs, :out_c]
    pos_out = jnp.zeros((num_graphs, 3), dtype=pos.dtype)
    batch_out = jnp.arange(num_graphs, dtype=jnp.int32)
    return pooled, pos_out, batch_out


def kernel(x, pos, batch, weight, bias):
    return _forward(x, pos, batch, weight, bias, NUM_GRAPHS_STATIC)


# speedup vs baseline: 11.8883x; 1.0611x over previous
import functools

import jax
import jax.numpy as jnp
from jax import lax
from jax.experimental import pallas as pl
from jax.experimental.pallas import tpu as pltpu


NUM_GRAPHS_STATIC = 64
_NEG_INF = float("-inf")


def _round_up(v, m):
    return ((v + m - 1) // m) * m


def _fused_kernel(offs_ref, x_ref, w_ref, b_ref, o_ref,
                  y_ref, bm_ref, acc_ref,
                  *, tm, rb, n_tiles, tiles_per_core, num_graphs):
    c = pl.program_id(0)
    i = pl.program_id(1)
    nb = tm // rb

    @pl.when(i == 0)
    def _init():
        acc_ref[...] = jnp.full(acc_ref.shape, _NEG_INF, dtype=acc_ref.dtype)

    y_ref[...] = lax.dot_general(
        x_ref[...].astype(jnp.bfloat16), w_ref[...].astype(jnp.bfloat16),
        dimension_numbers=(((1,), (1,)), ((), ())),
        preferred_element_type=jnp.float32,
    )

    for j in range(nb):
        bm_ref[j:j + 1, :] = y_ref[j * rb:(j + 1) * rb, :].max(
            axis=0, keepdims=True)

    tile_idx = jnp.minimum(c * tiles_per_core + i, n_tiles - 1)
    ts = tile_idx * tm
    te = ts + tm

    block_rows = lax.broadcasted_iota(jnp.int32, (nb, 1), 0)
    part_rows = lax.broadcasted_iota(jnp.int32, (rb, 1), 0)

    for g in range(num_graphs):
        lo = offs_ref[g]
        hi = offs_ref[g + 1]

        @pl.when(jnp.logical_and(hi > ts, lo < te))
        def _graph(g=g, lo=lo, hi=hi):
            a = jnp.maximum(lo, ts) - ts
            b = jnp.minimum(hi, te) - ts
            j_first = a // rb
            j_last = (b - 1) // rb
            full_lo = (a + rb - 1) // rb
            full_hi = b // rb

            bm_mask = jnp.logical_and(block_rows >= full_lo,
                                      block_rows < full_hi)
            cand = jnp.where(bm_mask, bm_ref[...], _NEG_INF).max(
                axis=0, keepdims=True)
            acc_ref[g:g + 1, :] = jnp.maximum(acc_ref[g:g + 1, :], cand)

            need_a = jnp.logical_or(
                a % rb != 0,
                jnp.logical_and(j_first == j_last, b % rb != 0))

            @pl.when(need_a)
            def _partial_a():
                blk = y_ref[pl.ds(j_first * rb, rb), :]
                rows = j_first * rb + part_rows
                m = jnp.logical_and(rows >= a, rows < b)
                pa = jnp.where(m, blk, _NEG_INF).max(axis=0, keepdims=True)
                acc_ref[g:g + 1, :] = jnp.maximum(acc_ref[g:g + 1, :], pa)

            need_b = jnp.logical_and(b % rb != 0, j_last > j_first)

            @pl.when(need_b)
            def _partial_b():
                blk = y_ref[pl.ds(j_last * rb, rb), :]
                rows = j_last * rb + part_rows
                m = jnp.logical_and(rows >= a, rows < b)
                pb = jnp.where(m, blk, _NEG_INF).max(axis=0, keepdims=True)
                acc_ref[g:g + 1, :] = jnp.maximum(acc_ref[g:g + 1, :], pb)

    @pl.when(i == pl.num_programs(1) - 1)
    def _finalize():
        o_ref[...] = (acc_ref[...] + b_ref[...]).astype(o_ref.dtype)


def _forward(x, pos, batch, weight, bias, num_graphs, *, tm=8192, rb=64):
    n, in_c = x.shape
    out_c = weight.shape[0]

    out_c_pad = _round_up(out_c, 128)
    g_pad = _round_up(num_graphs, 8)
    if out_c_pad != out_c:
        weight = jnp.pad(weight, ((0, out_c_pad - out_c), (0, 0)))
        bias = jnp.pad(bias, (0, out_c_pad - out_c))
    b2d = bias.reshape(1, out_c_pad).astype(jnp.float32)

    offsets = jnp.searchsorted(
        batch.astype(jnp.int32),
        jnp.arange(num_graphs + 1, dtype=jnp.int32)).astype(jnp.int32)

    tm_eff = min(max(rb, (tm // rb) * rb), _round_up(n, rb))
    n_tiles = pl.cdiv(n, tm_eff)
    num_cores = 2 if n_tiles >= 2 else 1
    tiles_per_core = pl.cdiv(n_tiles, num_cores)

    def x_map(c, i, offs):
        return (jnp.minimum(c * tiles_per_core + i, n_tiles - 1), 0)

    kernel_fn = functools.partial(
        _fused_kernel,
        tm=tm_eff, rb=rb, n_tiles=n_tiles, tiles_per_core=tiles_per_core,
        num_graphs=num_graphs)

    bytes_accessed = (x.size * x.dtype.itemsize
                      + weight.size * weight.dtype.itemsize
                      + (num_graphs + 1) * 4
                      + num_cores * g_pad * out_c_pad * 4)

    out = pl.pallas_call(
        kernel_fn,
        out_shape=jax.ShapeDtypeStruct((num_cores, g_pad, out_c_pad),
                                       jnp.float32),
        grid_spec=pltpu.PrefetchScalarGridSpec(
            num_scalar_prefetch=1,
            grid=(num_cores, tiles_per_core),
            in_specs=[
                pl.BlockSpec((tm_eff, in_c), x_map),
                pl.BlockSpec((out_c_pad, in_c), lambda c, i, offs: (0, 0)),
                pl.BlockSpec((1, out_c_pad), lambda c, i, offs: (0, 0)),
            ],
            out_specs=pl.BlockSpec((None, g_pad, out_c_pad),
                                   lambda c, i, offs: (c, 0, 0)),
            scratch_shapes=[
                pltpu.VMEM((tm_eff, out_c_pad), jnp.float32),
                pltpu.VMEM((tm_eff // rb, out_c_pad), jnp.float32),
                pltpu.VMEM((g_pad, out_c_pad), jnp.float32),
            ],
        ),
        compiler_params=pltpu.CompilerParams(
            dimension_semantics=("parallel", "arbitrary"),
            vmem_limit_bytes=48 * 1024 * 1024,
        ),
        cost_estimate=pl.CostEstimate(
            flops=2 * n * in_c * out_c_pad,
            transcendentals=0,
            bytes_accessed=bytes_accessed,
        ),
    )(offsets, x, weight, b2d)

    pooled = jnp.max(out, axis=0)[:num_graphs, :out_c]
    pos_out = jnp.zeros((num_graphs, 3), dtype=pos.dtype)
    batch_out = jnp.arange(num_graphs, dtype=jnp.int32)
    return pooled, pos_out, batch_out


def kernel(x, pos, batch, weight, bias):
    return _forward(x, pos, batch, weight, bias, NUM_GRAPHS_STATIC)


# speedup vs baseline: 11.9814x; 1.0078x over previous
import functools

import jax
import jax.numpy as jnp
from jax import lax
from jax.experimental import pallas as pl
from jax.experimental.pallas import tpu as pltpu


NUM_GRAPHS_STATIC = 64
_NEG_INF = float("-inf")


def _round_up(v, m):
    return ((v + m - 1) // m) * m


def _fused_kernel(offs_ref, x_ref, w_ref, b_ref, o_ref,
                  y_ref, bm_ref, acc_ref,
                  *, tm, rb, n_tiles, tiles_per_core, num_graphs):
    c = pl.program_id(0)
    i = pl.program_id(1)
    nb = tm // rb

    @pl.when(i == 0)
    def _init():
        acc_ref[...] = jnp.full(acc_ref.shape, _NEG_INF, dtype=acc_ref.dtype)

    wb = w_ref[...].astype(jnp.bfloat16)
    mc = 2048 if tm % 2048 == 0 else tm
    bpc = mc // rb
    for ci in range(tm // mc):
        r0 = ci * mc
        y_ref[r0:r0 + mc, :] = lax.dot_general(
            x_ref[r0:r0 + mc, :].astype(jnp.bfloat16), wb,
            dimension_numbers=(((1,), (1,)), ((), ())),
            preferred_element_type=jnp.float32,
        )
        for j in range(bpc):
            blk = ci * bpc + j
            bm_ref[blk:blk + 1, :] = y_ref[r0 + j * rb:r0 + (j + 1) * rb, :].max(
                axis=0, keepdims=True)

    tile_idx = jnp.minimum(c * tiles_per_core + i, n_tiles - 1)
    ts = tile_idx * tm
    te = ts + tm

    block_rows = lax.broadcasted_iota(jnp.int32, (nb, 1), 0)
    part_rows = lax.broadcasted_iota(jnp.int32, (rb, 1), 0)

    for g in range(num_graphs):
        lo = offs_ref[g]
        hi = offs_ref[g + 1]

        @pl.when(jnp.logical_and(hi > ts, lo < te))
        def _graph(g=g, lo=lo, hi=hi):
            a = jnp.maximum(lo, ts) - ts
            b = jnp.minimum(hi, te) - ts
            j_first = a // rb
            j_last = (b - 1) // rb
            full_lo = (a + rb - 1) // rb
            full_hi = b // rb

            bm_mask = jnp.logical_and(block_rows >= full_lo,
                                      block_rows < full_hi)
            cand = jnp.where(bm_mask, bm_ref[...], _NEG_INF).max(
                axis=0, keepdims=True)
            acc_ref[g:g + 1, :] = jnp.maximum(acc_ref[g:g + 1, :], cand)

            need_a = jnp.logical_or(
                a % rb != 0,
                jnp.logical_and(j_first == j_last, b % rb != 0))

            @pl.when(need_a)
            def _partial_a():
                blk = y_ref[pl.ds(j_first * rb, rb), :]
                rows = j_first * rb + part_rows
                m = jnp.logical_and(rows >= a, rows < b)
                pa = jnp.where(m, blk, _NEG_INF).max(axis=0, keepdims=True)
                acc_ref[g:g + 1, :] = jnp.maximum(acc_ref[g:g + 1, :], pa)

            need_b = jnp.logical_and(b % rb != 0, j_last > j_first)

            @pl.when(need_b)
            def _partial_b():
                blk = y_ref[pl.ds(j_last * rb, rb), :]
                rows = j_last * rb + part_rows
                m = jnp.logical_and(rows >= a, rows < b)
                pb = jnp.where(m, blk, _NEG_INF).max(axis=0, keepdims=True)
                acc_ref[g:g + 1, :] = jnp.maximum(acc_ref[g:g + 1, :], pb)

    @pl.when(i == pl.num_programs(1) - 1)
    def _finalize():
        o_ref[...] = (acc_ref[...] + b_ref[...]).astype(o_ref.dtype)


def _forward(x, pos, batch, weight, bias, num_graphs, *, tm=8192, rb=128):
    n, in_c = x.shape
    out_c = weight.shape[0]

    out_c_pad = _round_up(out_c, 128)
    g_pad = _round_up(num_graphs, 8)
    if out_c_pad != out_c:
        weight = jnp.pad(weight, ((0, out_c_pad - out_c), (0, 0)))
        bias = jnp.pad(bias, (0, out_c_pad - out_c))
    b2d = bias.reshape(1, out_c_pad).astype(jnp.float32)

    offsets = jnp.searchsorted(
        batch.astype(jnp.int32),
        jnp.arange(num_graphs + 1, dtype=jnp.int32)).astype(jnp.int32)

    tm_eff = min(max(rb, (tm // rb) * rb), _round_up(n, rb))
    n_tiles = pl.cdiv(n, tm_eff)
    num_cores = 2 if n_tiles >= 2 else 1
    tiles_per_core = pl.cdiv(n_tiles, num_cores)

    def x_map(c, i, offs):
        return (jnp.minimum(c * tiles_per_core + i, n_tiles - 1), 0)

    kernel_fn = functools.partial(
        _fused_kernel,
        tm=tm_eff, rb=rb, n_tiles=n_tiles, tiles_per_core=tiles_per_core,
        num_graphs=num_graphs)

    bytes_accessed = (x.size * x.dtype.itemsize
                      + weight.size * weight.dtype.itemsize
                      + (num_graphs + 1) * 4
                      + num_cores * g_pad * out_c_pad * 4)

    out = pl.pallas_call(
        kernel_fn,
        out_shape=jax.ShapeDtypeStruct((num_cores, g_pad, out_c_pad),
                                       jnp.float32),
        grid_spec=pltpu.PrefetchScalarGridSpec(
            num_scalar_prefetch=1,
            grid=(num_cores, tiles_per_core),
            in_specs=[
                pl.BlockSpec((tm_eff, in_c), x_map),
                pl.BlockSpec((out_c_pad, in_c), lambda c, i, offs: (0, 0)),
                pl.BlockSpec((1, out_c_pad), lambda c, i, offs: (0, 0)),
            ],
            out_specs=pl.BlockSpec((None, g_pad, out_c_pad),
                                   lambda c, i, offs: (c, 0, 0)),
            scratch_shapes=[
                pltpu.VMEM((tm_eff, out_c_pad), jnp.float32),
                pltpu.VMEM((tm_eff // rb, out_c_pad), jnp.float32),
                pltpu.VMEM((g_pad, out_c_pad), jnp.float32),
            ],
        ),
        compiler_params=pltpu.CompilerParams(
            dimension_semantics=("parallel", "arbitrary"),
            vmem_limit_bytes=48 * 1024 * 1024,
        ),
        cost_estimate=pl.CostEstimate(
            flops=2 * n * in_c * out_c_pad,
            transcendentals=0,
            bytes_accessed=bytes_accessed,
        ),
    )(offsets, x, weight, b2d)

    pooled = jnp.max(out, axis=0)[:num_graphs, :out_c]
    pos_out = jnp.zeros((num_graphs, 3), dtype=pos.dtype)
    batch_out = jnp.arange(num_graphs, dtype=jnp.int32)
    return pooled, pos_out, batch_out


def kernel(x, pos, batch, weight, bias):
    return _forward(x, pos, batch, weight, bias, NUM_GRAPHS_STATIC)
